# baseline (speedup 1.0000x reference)
import functools

import jax
import jax.numpy as jnp
from jax import lax
from jax.experimental import pallas as pl
from jax.experimental.pallas import tpu as pltpu
from jax.experimental.pallas import tpu_sc as plsc

N = 100000
E = 3200000
SHEET_COND = 0.01
SHEET_EXP = 1.25
CHAN_COND = 0.1
CHAN_EXP = 3.0
BED_STEP = 0.1
CAV_SPACING = 2.0
CLOSURE = 5e-25
PMC = 7.5e-08
CP = 4220.0
RHO_W = 1000.0
RHO_I = 917.0
G = 9.81
SEC_PER_A = 31556926.0
LATENT = 334000.0
RWG = RHO_W * G
RIG = RHO_I * G

NC = 2
NS = 16
NW = NC * NS

CH = 128
RB = 8
UE = RB * CH
N_UNITS = E // UE
BASE_UNITS = N_UNITS // NW
EXTRA = N_UNITS - BASE_UNITS * NW

NSL = 6240
NTAIL = N - NS * NSL
NTB = NS * NSL

WSL = 3120
WTAIL = N - NW * WSL
WTB = NW * WSL


def _rsqrt(x):
    i = lax.bitcast_convert_type(x, jnp.int32)
    i = jnp.int32(0x5F3759DF) - (i >> 1)
    y = lax.bitcast_convert_type(i, jnp.float32)
    for _ in range(2):
        y = y * (1.5 - 0.5 * x * y * y)
    return y


def _edge_math(pth, ptt, hne_h, hne_t, wph, wpt, s_ch, u_sl):
    gh = lax.bitcast_convert_type(hne_h, jnp.int32)
    gt = lax.bitcast_convert_type(hne_t, jnp.int32)
    hh = lax.bitcast_convert_type(gh << 16, jnp.float32)
    ht = lax.bitcast_convert_type(gt << 16, jnp.float32)
    neh = lax.bitcast_convert_type(gh & jnp.int32(-65536), jnp.float32)
    net = lax.bitcast_convert_type(gt & jnp.int32(-65536), jnp.float32)
    grad = pth - ptt
    absg = jnp.abs(grad) + 1e-8
    hl = 0.5 * (hh + ht)
    hs = jnp.maximum(hl, 1e-30)
    hp = hl * _rsqrt(_rsqrt(hs))
    sheet_q = (-SHEET_COND) * hp * _rsqrt(absg) * grad
    chan_q = (-CHAN_COND) * (s_ch * s_ch * s_ch) * grad
    diss = jnp.abs(chan_q * grad) + jnp.abs(CAV_SPACING * sheet_q * grad)
    pgrad = wph - wpt
    cond = (s_ch > 0) | ((pgrad * sheet_q) > 0)
    totq = jnp.where(cond, chan_q + CAV_SPACING * sheet_q, chan_q)
    sens = (-PMC * CP * RHO_W) * totq * pgrad
    nl = 0.5 * (neh + net)
    nlc = jnp.maximum(nl, 0.0)
    ccl = CLOSURE * s_ch * (nlc * nlc * nlc)
    melt = (diss - sens) * (1.0 / (RHO_I * LATENT))
    dsdt = melt - ccl
    aslide = jnp.abs(u_sl) * (1.0 / SEC_PER_A)
    return dsdt, aslide


def _edge_body(pot_hbm, h_hbm, bed_hbm, ice_hbm, chan_hbm, slid_hbm, tail_hbm, head_hbm,
               dq_out, sl0_out, sl1_out, dg0_out, dg1_out,
               hne_sh, wp_sh, slide_sh, deg_sh,
               b1, b2, b3,
               ihb, itb, sv, uv, gph, gpt, ghh, ght, gwh, gwt, dq, sl, ones,
               seml, semg, semh, sems):
    c = lax.axis_index("c")
    s = lax.axis_index("s")
    w = s * NC + c

    def _stage(nb, nsl, iters):
        pltpu.sync_copy(pot_hbm.at[pl.ds(nb, nsl)], b3.at[pl.ds(0, nsl)])
        pltpu.sync_copy(bed_hbm.at[pl.ds(nb, nsl)], b1.at[pl.ds(0, nsl)])
        pltpu.sync_copy(ice_hbm.at[pl.ds(nb, nsl)], b2.at[pl.ds(0, nsl)])

        def nbody(i, carry):
            dsl = pl.ds(pl.multiple_of(i * 16, 16), 16)
            p = b3[dsl]
            bp = RWG * b1[dsl]
            icv = b2[dsl]
            b1[dsl] = p - bp
            b2[dsl] = bp + RIG * icv - p
            return carry

        lax.fori_loop(0, iters, nbody, 0)
        pltpu.sync_copy(b1.at[pl.ds(0, nsl)], wp_sh.at[pl.ds(nb, nsl)])
        pltpu.sync_copy(h_hbm.at[pl.ds(nb, nsl)], b3.at[pl.ds(0, nsl)])

        def pbody(i, carry):
            dsl = pl.ds(pl.multiple_of(i * 16, 16), 16)
            neb = lax.bitcast_convert_type(b2[dsl], jnp.int32)
            hb = lax.bitcast_convert_type(b3[dsl], jnp.int32)
            pk = (neb & jnp.int32(-65536)) | (hb >> 16)
            b2[dsl] = lax.bitcast_convert_type(pk, jnp.float32)
            return carry

        lax.fori_loop(0, iters, pbody, 0)
        pltpu.sync_copy(b2.at[pl.ds(0, nsl)], hne_sh.at[pl.ds(nb, nsl)])

        def zbody(i, carry):
            dsl = pl.ds(pl.multiple_of(i * 16, 16), 16)
            b1[dsl] = jnp.zeros((16,), jnp.float32)
            return carry

        lax.fori_loop(0, iters, zbody, 0)
        pltpu.sync_copy(b1.at[pl.ds(0, nsl)], slide_sh.at[pl.ds(nb, nsl)])
        pltpu.sync_copy(b1.at[pl.ds(0, nsl)], deg_sh.at[pl.ds(nb, nsl)])

    _stage(pl.multiple_of(s * NSL, 32), NSL, NSL // 16)

    @pl.when(s == 0)
    def _():
        _stage(NTB, NTAIL, NTAIL // 16)

    for i in range(UE // 16):
        ones[pl.ds(i * 16, 16)] = jnp.ones((16,), jnp.float32)

    plsc.subcore_barrier()

    n_iters = jnp.where(w < EXTRA, BASE_UNITS + 1, BASE_UNITS)
    start = BASE_UNITS * w + jnp.minimum(w, EXTRA)

    def _po(p):
        return pl.ds(pl.multiple_of(p * UE, UE), UE)

    def _fire_linear(j, p):
        base = pl.multiple_of((start + j) * UE, UE)
        pltpu.async_copy(chan_hbm.at[pl.ds(base, UE)], sv.at[_po(p)], seml)
        pltpu.async_copy(slid_hbm.at[pl.ds(base, UE)], uv.at[_po(p)], seml)
        pltpu.async_copy(head_hbm.at[pl.ds(base, UE)], ihb.at[_po(p)], seml)
        pltpu.async_copy(tail_hbm.at[pl.ds(base, UE)], itb.at[_po(p)], seml)

    def _drain_linear(p):
        pltpu.make_async_copy(chan_hbm.at[pl.ds(0, UE)], sv.at[_po(p)], seml).wait()
        pltpu.make_async_copy(slid_hbm.at[pl.ds(0, UE)], uv.at[_po(p)], seml).wait()
        pltpu.make_async_copy(head_hbm.at[pl.ds(0, UE)], ihb.at[_po(p)], seml).wait()
        pltpu.make_async_copy(tail_hbm.at[pl.ds(0, UE)], itb.at[_po(p)], seml).wait()

    def _gather_list(p):
        ihr = ihb.at[_po(p)]
        itr = itb.at[_po(p)]
        hb = [(pot_hbm.at[ihr], gph.at[_po(p)]),
              (pot_hbm.at[itr], gpt.at[_po(p)])]
        sp = [(hne_sh.at[ihr], ghh.at[_po(p)]),
              (hne_sh.at[itr], ght.at[_po(p)]),
              (wp_sh.at[ihr], gwh.at[_po(p)]),
              (wp_sh.at[itr], gwt.at[_po(p)])]
        return sp, hb

    def _fire_gathers(p):
        sp, hb = _gather_list(p)
        for src, dst in hb:
            pltpu.async_copy(src, dst, semh.at[p])
        for src, dst in sp:
            pltpu.async_copy(src, dst, semg.at[p])

    def _drain_gathers(p):
        sp, hb = _gather_list(p)
        for src, dst in sp:
            pltpu.make_async_copy(src, dst, semg.at[p]).wait()
        for src, dst in hb:
            pltpu.make_async_copy(src, dst, semh.at[p]).wait()

    _fire_linear(0, 0)
    _drain_linear(0)
    _fire_gathers(0)
    _fire_linear(1, 1)

    def ebody(j, carry):
        p = lax.rem(j, 2)
        q = 1 - p
        base = pl.multiple_of((start + j) * UE, UE)

        @pl.when(j + 1 < n_iters)
        def _():
            _drain_linear(q)
            _fire_gathers(q)

        _drain_gathers(p)
        pb = pl.multiple_of(p * UE, UE)
        for i in range(UE // 16):
            dsl = pl.ds(pb + i * 16, 16)
            dsdt, aslide = _edge_math(gph[dsl], gpt[dsl],
                                      ghh[dsl], ght[dsl],
                                      gwh[dsl], gwt[dsl],
                                      sv[dsl], uv[dsl])
            dq[dsl] = dsdt
            sl[dsl] = aslide
        pltpu.async_copy(sl.at[_po(p)], slide_sh.at[ihb.at[_po(p)]], sems, add=True)
        pltpu.async_copy(sl.at[_po(p)], slide_sh.at[itb.at[_po(p)]], sems, add=True)
        pltpu.async_copy(ones, deg_sh.at[ihb.at[_po(p)]], sems, add=True)
        pltpu.async_copy(ones, deg_sh.at[itb.at[_po(p)]], sems, add=True)
        ocp = pltpu.async_copy(dq.at[_po(p)], dq_out.at[pl.ds(base, UE)], seml)
        pltpu.make_async_copy(sl.at[_po(p)], slide_sh.at[ihb.at[_po(p)]], sems).wait()
        pltpu.make_async_copy(sl.at[_po(p)], slide_sh.at[itb.at[_po(p)]], sems).wait()
        pltpu.make_async_copy(ones, deg_sh.at[ihb.at[_po(p)]], sems).wait()
        pltpu.make_async_copy(ones, deg_sh.at[itb.at[_po(p)]], sems).wait()
        ocp.wait()

        @pl.when(j + 2 < n_iters)
        def _():
            _fire_linear(j + 2, p)

        return carry

    lax.fori_loop(0, n_iters, ebody, 0)

    plsc.subcore_barrier()

    def _wb(nb, nsl, slide_out, deg_out):
        pltpu.sync_copy(slide_sh.at[pl.ds(nb, nsl)], b1.at[pl.ds(0, nsl)])
        pltpu.sync_copy(b1.at[pl.ds(0, nsl)], slide_out.at[pl.ds(nb, nsl)])
        pltpu.sync_copy(deg_sh.at[pl.ds(nb, nsl)], b2.at[pl.ds(0, nsl)])
        pltpu.sync_copy(b2.at[pl.ds(0, nsl)], deg_out.at[pl.ds(nb, nsl)])

    nb_main = pl.multiple_of(s * NSL, 32)

    @pl.when(c == 0)
    def _():
        _wb(nb_main, NSL, sl0_out, dg0_out)

    @pl.when(c == 1)
    def _():
        _wb(nb_main, NSL, sl1_out, dg1_out)

    @pl.when((s == 0) & (c == 0))
    def _():
        _wb(NTB, NTAIL, sl0_out, dg0_out)

    @pl.when((s == 0) & (c == 1))
    def _():
        _wb(NTB, NTAIL, sl1_out, dg1_out)


def _node_body(pot_hbm, h_hbm, bed_hbm, ice_hbm, sl0_hbm, sl1_hbm, dg0_hbm, dg1_hbm,
               dh_out,
               potb, hb, bedb, iceb, sp0, sp1, dp0, dp1, dhb):
    c = lax.axis_index("c")
    s = lax.axis_index("s")
    w = s * NC + c

    def _run(nb, nsl, iters):
        pltpu.sync_copy(pot_hbm.at[pl.ds(nb, nsl)], potb.at[pl.ds(0, nsl)])
        pltpu.sync_copy(h_hbm.at[pl.ds(nb, nsl)], hb.at[pl.ds(0, nsl)])
        pltpu.sync_copy(bed_hbm.at[pl.ds(nb, nsl)], bedb.at[pl.ds(0, nsl)])
        pltpu.sync_copy(ice_hbm.at[pl.ds(nb, nsl)], iceb.at[pl.ds(0, nsl)])
        pltpu.sync_copy(sl0_hbm.at[pl.ds(nb, nsl)], sp0.at[pl.ds(0, nsl)])
        pltpu.sync_copy(sl1_hbm.at[pl.ds(nb, nsl)], sp1.at[pl.ds(0, nsl)])
        pltpu.sync_copy(dg0_hbm.at[pl.ds(nb, nsl)], dp0.at[pl.ds(0, nsl)])
        pltpu.sync_copy(dg1_hbm.at[pl.ds(nb, nsl)], dp1.at[pl.ds(0, nsl)])

        def nbody(i, carry):
            dsl = pl.ds(pl.multiple_of(i * 16, 16), 16)
            p = potb[dsl]
            h = hb[dsl]
            ne = RWG * bedb[dsl] + RIG * iceb[dsl] - p
            nec = jnp.maximum(ne, 0.0)
            scl = CLOSURE * h * (nec * nec * nec)
            dg = dp0[dsl] + dp1[dsl]
            sn = (sp0[dsl] + sp1[dsl]) / jnp.maximum(dg, 1.0)
            opening = jnp.where(h < BED_STEP,
                                sn * (BED_STEP - h) * (1.0 / CAV_SPACING), 0.0)
            dhb[dsl] = opening - scl
            return carry

        lax.fori_loop(0, iters, nbody, 0)
        pltpu.sync_copy(dhb.at[pl.ds(0, nsl)], dh_out.at[pl.ds(nb, nsl)])

    _run(pl.multiple_of(w * WSL, 16), WSL, WSL // 16)

    @pl.when(w == 0)
    def _():
        _run(WTB, WTAIL, WTAIL // 16)


_MESH = plsc.VectorSubcoreMesh(core_axis_name="c", subcore_axis_name="s")

_edge_kernel = functools.partial(
    pl.kernel,
    out_type=(jax.ShapeDtypeStruct((E,), jnp.float32),
              jax.ShapeDtypeStruct((N,), jnp.float32),
              jax.ShapeDtypeStruct((N,), jnp.float32),
              jax.ShapeDtypeStruct((N,), jnp.float32),
              jax.ShapeDtypeStruct((N,), jnp.float32)),
    mesh=_MESH,
    scratch_types=(
        pltpu.VMEM_SHARED((N,), jnp.float32),
        pltpu.VMEM_SHARED((N,), jnp.float32),
        pltpu.VMEM_SHARED((N,), jnp.float32),
        pltpu.VMEM_SHARED((N,), jnp.float32),
        pltpu.VMEM((NSL,), jnp.float32),
        pltpu.VMEM((NSL,), jnp.float32),
        pltpu.VMEM((NSL,), jnp.float32),
        pltpu.VMEM((2 * UE,), jnp.int32),
        pltpu.VMEM((2 * UE,), jnp.int32),
        pltpu.VMEM((2 * UE,), jnp.float32),
        pltpu.VMEM((2 * UE,), jnp.float32),
        pltpu.VMEM((2 * UE,), jnp.float32),
        pltpu.VMEM((2 * UE,), jnp.float32),
        pltpu.VMEM((2 * UE,), jnp.float32),
        pltpu.VMEM((2 * UE,), jnp.float32),
        pltpu.VMEM((2 * UE,), jnp.float32),
        pltpu.VMEM((2 * UE,), jnp.float32),
        pltpu.VMEM((2 * UE,), jnp.float32),
        pltpu.VMEM((2 * UE,), jnp.float32),
        pltpu.VMEM((UE,), jnp.float32),
        pltpu.SemaphoreType.DMA,
        pltpu.SemaphoreType.DMA((2,)),
        pltpu.SemaphoreType.DMA((2,)),
        pltpu.SemaphoreType.DMA,
    ),
)(_edge_body)

_node_kernel = functools.partial(
    pl.kernel,
    out_type=jax.ShapeDtypeStruct((N,), jnp.float32),
    mesh=_MESH,
    scratch_types=tuple([pltpu.VMEM((WSL,), jnp.float32)] * 9),
)(_node_body)


def kernel(potential, sheet_thickness, channel_size, sliding_velocity,
           bedrock_elevation, ice_thickness, edge_index):
    tail = edge_index[0]
    head = edge_index[1]
    dsdt, sl0, sl1, dg0, dg1 = _edge_kernel(
        potential, sheet_thickness, bedrock_elevation, ice_thickness,
        channel_size, sliding_velocity, tail, head)
    dhdt = _node_kernel(potential, sheet_thickness, bedrock_elevation,
                        ice_thickness, sl0, sl1, dg0, dg1)
    return jnp.concatenate([dhdt, dsdt])


# speedup vs baseline: 1.2816x; 1.2816x over previous
import functools

import jax
import jax.numpy as jnp
from jax import lax
from jax.experimental import pallas as pl
from jax.experimental.pallas import tpu as pltpu
from jax.experimental.pallas import tpu_sc as plsc

N = 100000
E = 3200000
SHEET_COND = 0.01
SHEET_EXP = 1.25
CHAN_COND = 0.1
CHAN_EXP = 3.0
BED_STEP = 0.1
CAV_SPACING = 2.0
CLOSURE = 5e-25
PMC = 7.5e-08
CP = 4220.0
RHO_W = 1000.0
RHO_I = 917.0
G = 9.81
SEC_PER_A = 31556926.0
LATENT = 334000.0
RWG = RHO_W * G
RIG = RHO_I * G

NC = 2
NS = 16
NW = NC * NS

CH = 128
RB = 4
UE = RB * CH
N_UNITS = E // UE
BASE_UNITS = N_UNITS // NW
EXTRA = N_UNITS - BASE_UNITS * NW

NSL = 6240
NTAIL = N - NS * NSL
NTB = NS * NSL

WSL = 3120
WTAIL = N - NW * WSL
WTB = NW * WSL


def _rsqrt(x):
    i = lax.bitcast_convert_type(x, jnp.int32)
    i = jnp.int32(0x5F3759DF) - (i >> 1)
    y = lax.bitcast_convert_type(i, jnp.float32)
    for _ in range(2):
        y = y * (1.5 - 0.5 * x * y * y)
    return y


def _edge_math(pth, ptt, hne_h, hne_t, wph, wpt, s_ch, u_sl):
    gh = lax.bitcast_convert_type(hne_h, jnp.int32)
    gt = lax.bitcast_convert_type(hne_t, jnp.int32)
    hh = lax.bitcast_convert_type(gh << 16, jnp.float32)
    ht = lax.bitcast_convert_type(gt << 16, jnp.float32)
    neh = lax.bitcast_convert_type(gh & jnp.int32(-65536), jnp.float32)
    net = lax.bitcast_convert_type(gt & jnp.int32(-65536), jnp.float32)
    grad = pth - ptt
    absg = jnp.abs(grad) + 1e-8
    hl = 0.5 * (hh + ht)
    hs = jnp.maximum(hl, 1e-30)
    hp = hl * _rsqrt(_rsqrt(hs))
    sheet_q = (-SHEET_COND) * hp * _rsqrt(absg) * grad
    chan_q = (-CHAN_COND) * (s_ch * s_ch * s_ch) * grad
    diss = jnp.abs(chan_q * grad) + jnp.abs(CAV_SPACING * sheet_q * grad)
    pgrad = wph - wpt
    cond = (s_ch > 0) | ((pgrad * sheet_q) > 0)
    totq = jnp.where(cond, chan_q + CAV_SPACING * sheet_q, chan_q)
    sens = (-PMC * CP * RHO_W) * totq * pgrad
    nl = 0.5 * (neh + net)
    nlc = jnp.maximum(nl, 0.0)
    ccl = CLOSURE * s_ch * (nlc * nlc * nlc)
    melt = (diss - sens) * (1.0 / (RHO_I * LATENT))
    dsdt = melt - ccl
    aslide = jnp.abs(u_sl) * (1.0 / SEC_PER_A)
    return dsdt, aslide


def _edge_body(pot_hbm, h_hbm, bed_hbm, ice_hbm, chan_hbm, slid_hbm, tail_hbm, head_hbm,
               dq_out, sl0_out, sl1_out, dg0_out, dg1_out,
               hne_sh, wp_sh, slide_sh, deg_sh,
               b1, b2, b3,
               ihb, itb, sv, uv, gph, gpt, ghh, ght, gwh, gwt, dq, sl, ones,
               seml, semg, semh, sems):
    c = lax.axis_index("c")
    s = lax.axis_index("s")
    w = s * NC + c

    def _stage(nb, nsl, iters):
        pltpu.sync_copy(pot_hbm.at[pl.ds(nb, nsl)], b3.at[pl.ds(0, nsl)])
        pltpu.sync_copy(bed_hbm.at[pl.ds(nb, nsl)], b1.at[pl.ds(0, nsl)])
        pltpu.sync_copy(ice_hbm.at[pl.ds(nb, nsl)], b2.at[pl.ds(0, nsl)])

        def nbody(i, carry):
            dsl = pl.ds(pl.multiple_of(i * 16, 16), 16)
            p = b3[dsl]
            bp = RWG * b1[dsl]
            icv = b2[dsl]
            b1[dsl] = p - bp
            b2[dsl] = bp + RIG * icv - p
            return carry

        lax.fori_loop(0, iters, nbody, 0)
        pltpu.sync_copy(b1.at[pl.ds(0, nsl)], wp_sh.at[pl.ds(nb, nsl)])
        pltpu.sync_copy(h_hbm.at[pl.ds(nb, nsl)], b3.at[pl.ds(0, nsl)])

        def pbody(i, carry):
            dsl = pl.ds(pl.multiple_of(i * 16, 16), 16)
            neb = lax.bitcast_convert_type(b2[dsl], jnp.int32)
            hb = lax.bitcast_convert_type(b3[dsl], jnp.int32)
            pk = (neb & jnp.int32(-65536)) | (hb >> 16)
            b2[dsl] = lax.bitcast_convert_type(pk, jnp.float32)
            return carry

        lax.fori_loop(0, iters, pbody, 0)
        pltpu.sync_copy(b2.at[pl.ds(0, nsl)], hne_sh.at[pl.ds(nb, nsl)])

        def zbody(i, carry):
            dsl = pl.ds(pl.multiple_of(i * 16, 16), 16)
            b1[dsl] = jnp.zeros((16,), jnp.float32)
            return carry

        lax.fori_loop(0, iters, zbody, 0)
        pltpu.sync_copy(b1.at[pl.ds(0, nsl)], slide_sh.at[pl.ds(nb, nsl)])
        pltpu.sync_copy(b1.at[pl.ds(0, nsl)], deg_sh.at[pl.ds(nb, nsl)])

    _stage(pl.multiple_of(s * NSL, 32), NSL, NSL // 16)

    @pl.when(s == 0)
    def _():
        _stage(NTB, NTAIL, NTAIL // 16)

    for i in range(UE // 16):
        ones[pl.ds(i * 16, 16)] = jnp.ones((16,), jnp.float32)

    plsc.subcore_barrier()

    n_iters = jnp.where(w < EXTRA, BASE_UNITS + 1, BASE_UNITS)
    start = BASE_UNITS * w + jnp.minimum(w, EXTRA)

    def _po(p):
        return pl.ds(pl.multiple_of(p * UE, UE), UE)

    def _fire_linear(j, p):
        base = pl.multiple_of((start + j) * UE, UE)
        pltpu.async_copy(chan_hbm.at[pl.ds(base, UE)], sv.at[_po(p)], seml)
        pltpu.async_copy(slid_hbm.at[pl.ds(base, UE)], uv.at[_po(p)], seml)
        pltpu.async_copy(head_hbm.at[pl.ds(base, UE)], ihb.at[_po(p)], seml)
        pltpu.async_copy(tail_hbm.at[pl.ds(base, UE)], itb.at[_po(p)], seml)

    def _drain_linear(p):
        pltpu.make_async_copy(chan_hbm.at[pl.ds(0, UE)], sv.at[_po(p)], seml).wait()
        pltpu.make_async_copy(slid_hbm.at[pl.ds(0, UE)], uv.at[_po(p)], seml).wait()
        pltpu.make_async_copy(head_hbm.at[pl.ds(0, UE)], ihb.at[_po(p)], seml).wait()
        pltpu.make_async_copy(tail_hbm.at[pl.ds(0, UE)], itb.at[_po(p)], seml).wait()

    def _gather_list(p):
        ihr = ihb.at[_po(p)]
        itr = itb.at[_po(p)]
        hb = [(pot_hbm.at[ihr], gph.at[_po(p)]),
              (pot_hbm.at[itr], gpt.at[_po(p)])]
        sp = [(hne_sh.at[ihr], ghh.at[_po(p)]),
              (hne_sh.at[itr], ght.at[_po(p)]),
              (wp_sh.at[ihr], gwh.at[_po(p)]),
              (wp_sh.at[itr], gwt.at[_po(p)])]
        return sp, hb

    def _fire_gathers(p):
        sp, hb = _gather_list(p)
        for src, dst in hb:
            pltpu.async_copy(src, dst, semh.at[p])
        for src, dst in sp:
            pltpu.async_copy(src, dst, semg.at[p])

    def _drain_gathers(p):
        sp, hb = _gather_list(p)
        for src, dst in sp:
            pltpu.make_async_copy(src, dst, semg.at[p]).wait()
        for src, dst in hb:
            pltpu.make_async_copy(src, dst, semh.at[p]).wait()

    _fire_linear(0, 0)
    _drain_linear(0)
    _fire_gathers(0)
    _fire_linear(1, 1)

    def ebody(j, carry):
        p = lax.rem(j, 2)
        q = 1 - p
        base = pl.multiple_of((start + j) * UE, UE)

        @pl.when(j + 1 < n_iters)
        def _():
            _drain_linear(q)
            _fire_gathers(q)

        _drain_gathers(p)
        pb = pl.multiple_of(p * UE, UE)
        for i in range(UE // 16):
            dsl = pl.ds(pb + i * 16, 16)
            dsdt, aslide = _edge_math(gph[dsl], gpt[dsl],
                                      ghh[dsl], ght[dsl],
                                      gwh[dsl], gwt[dsl],
                                      sv[dsl], uv[dsl])
            dq[dsl] = dsdt
            sl[dsl] = aslide
        pltpu.async_copy(sl.at[_po(p)], slide_sh.at[ihb.at[_po(p)]], sems, add=True)
        pltpu.async_copy(sl.at[_po(p)], slide_sh.at[itb.at[_po(p)]], sems, add=True)
        pltpu.async_copy(ones, deg_sh.at[ihb.at[_po(p)]], sems, add=True)
        pltpu.async_copy(ones, deg_sh.at[itb.at[_po(p)]], sems, add=True)
        ocp = pltpu.async_copy(dq.at[_po(p)], dq_out.at[pl.ds(base, UE)], seml)
        pltpu.make_async_copy(sl.at[_po(p)], slide_sh.at[ihb.at[_po(p)]], sems).wait()
        pltpu.make_async_copy(sl.at[_po(p)], slide_sh.at[itb.at[_po(p)]], sems).wait()
        pltpu.make_async_copy(ones, deg_sh.at[ihb.at[_po(p)]], sems).wait()
        pltpu.make_async_copy(ones, deg_sh.at[itb.at[_po(p)]], sems).wait()
        ocp.wait()

        @pl.when(j + 2 < n_iters)
        def _():
            _fire_linear(j + 2, p)

        return carry

    lax.fori_loop(0, n_iters, ebody, 0)

    plsc.subcore_barrier()

    def _wb(nb, nsl, slide_out, deg_out):
        pltpu.sync_copy(slide_sh.at[pl.ds(nb, nsl)], b1.at[pl.ds(0, nsl)])
        pltpu.sync_copy(b1.at[pl.ds(0, nsl)], slide_out.at[pl.ds(nb, nsl)])
        pltpu.sync_copy(deg_sh.at[pl.ds(nb, nsl)], b2.at[pl.ds(0, nsl)])
        pltpu.sync_copy(b2.at[pl.ds(0, nsl)], deg_out.at[pl.ds(nb, nsl)])

    nb_main = pl.multiple_of(s * NSL, 32)

    @pl.when(c == 0)
    def _():
        _wb(nb_main, NSL, sl0_out, dg0_out)

    @pl.when(c == 1)
    def _():
        _wb(nb_main, NSL, sl1_out, dg1_out)

    @pl.when((s == 0) & (c == 0))
    def _():
        _wb(NTB, NTAIL, sl0_out, dg0_out)

    @pl.when((s == 0) & (c == 1))
    def _():
        _wb(NTB, NTAIL, sl1_out, dg1_out)


def _node_body(pot_hbm, h_hbm, bed_hbm, ice_hbm, sl0_hbm, sl1_hbm, dg0_hbm, dg1_hbm,
               dh_out,
               potb, hb, bedb, iceb, sp0, sp1, dp0, dp1, dhb):
    c = lax.axis_index("c")
    s = lax.axis_index("s")
    w = s * NC + c

    def _run(nb, nsl, iters):
        pltpu.sync_copy(pot_hbm.at[pl.ds(nb, nsl)], potb.at[pl.ds(0, nsl)])
        pltpu.sync_copy(h_hbm.at[pl.ds(nb, nsl)], hb.at[pl.ds(0, nsl)])
        pltpu.sync_copy(bed_hbm.at[pl.ds(nb, nsl)], bedb.at[pl.ds(0, nsl)])
        pltpu.sync_copy(ice_hbm.at[pl.ds(nb, nsl)], iceb.at[pl.ds(0, nsl)])
        pltpu.sync_copy(sl0_hbm.at[pl.ds(nb, nsl)], sp0.at[pl.ds(0, nsl)])
        pltpu.sync_copy(sl1_hbm.at[pl.ds(nb, nsl)], sp1.at[pl.ds(0, nsl)])
        pltpu.sync_copy(dg0_hbm.at[pl.ds(nb, nsl)], dp0.at[pl.ds(0, nsl)])
        pltpu.sync_copy(dg1_hbm.at[pl.ds(nb, nsl)], dp1.at[pl.ds(0, nsl)])

        def nbody(i, carry):
            dsl = pl.ds(pl.multiple_of(i * 16, 16), 16)
            p = potb[dsl]
            h = hb[dsl]
            ne = RWG * bedb[dsl] + RIG * iceb[dsl] - p
            nec = jnp.maximum(ne, 0.0)
            scl = CLOSURE * h * (nec * nec * nec)
            dg = dp0[dsl] + dp1[dsl]
            sn = (sp0[dsl] + sp1[dsl]) / jnp.maximum(dg, 1.0)
            opening = jnp.where(h < BED_STEP,
                                sn * (BED_STEP - h) * (1.0 / CAV_SPACING), 0.0)
            dhb[dsl] = opening - scl
            return carry

        lax.fori_loop(0, iters, nbody, 0)
        pltpu.sync_copy(dhb.at[pl.ds(0, nsl)], dh_out.at[pl.ds(nb, nsl)])

    _run(pl.multiple_of(w * WSL, 16), WSL, WSL // 16)

    @pl.when(w == 0)
    def _():
        _run(WTB, WTAIL, WTAIL // 16)


_MESH = plsc.VectorSubcoreMesh(core_axis_name="c", subcore_axis_name="s")

_edge_kernel = functools.partial(
    pl.kernel,
    out_type=(jax.ShapeDtypeStruct((E,), jnp.float32),
              jax.ShapeDtypeStruct((N,), jnp.float32),
              jax.ShapeDtypeStruct((N,), jnp.float32),
              jax.ShapeDtypeStruct((N,), jnp.float32),
              jax.ShapeDtypeStruct((N,), jnp.float32)),
    mesh=_MESH,
    scratch_types=(
        pltpu.VMEM_SHARED((N,), jnp.float32),
        pltpu.VMEM_SHARED((N,), jnp.float32),
        pltpu.VMEM_SHARED((N,), jnp.float32),
        pltpu.VMEM_SHARED((N,), jnp.float32),
        pltpu.VMEM((NSL,), jnp.float32),
        pltpu.VMEM((NSL,), jnp.float32),
        pltpu.VMEM((NSL,), jnp.float32),
        pltpu.VMEM((2 * UE,), jnp.int32),
        pltpu.VMEM((2 * UE,), jnp.int32),
        pltpu.VMEM((2 * UE,), jnp.float32),
        pltpu.VMEM((2 * UE,), jnp.float32),
        pltpu.VMEM((2 * UE,), jnp.float32),
        pltpu.VMEM((2 * UE,), jnp.float32),
        pltpu.VMEM((2 * UE,), jnp.float32),
        pltpu.VMEM((2 * UE,), jnp.float32),
        pltpu.VMEM((2 * UE,), jnp.float32),
        pltpu.VMEM((2 * UE,), jnp.float32),
        pltpu.VMEM((2 * UE,), jnp.float32),
        pltpu.VMEM((2 * UE,), jnp.float32),
        pltpu.VMEM((UE,), jnp.float32),
        pltpu.SemaphoreType.DMA,
        pltpu.SemaphoreType.DMA((2,)),
        pltpu.SemaphoreType.DMA((2,)),
        pltpu.SemaphoreType.DMA,
    ),
)(_edge_body)

_node_kernel = functools.partial(
    pl.kernel,
    out_type=jax.ShapeDtypeStruct((N,), jnp.float32),
    mesh=_MESH,
    scratch_types=tuple([pltpu.VMEM((WSL,), jnp.float32)] * 9),
)(_node_body)


def kernel(potential, sheet_thickness, channel_size, sliding_velocity,
           bedrock_elevation, ice_thickness, edge_index):
    tail = edge_index[0]
    head = edge_index[1]
    dsdt, sl0, sl1, dg0, dg1 = _edge_kernel(
        potential, sheet_thickness, bedrock_elevation, ice_thickness,
        channel_size, sliding_velocity, tail, head)
    dhdt = _node_kernel(potential, sheet_thickness, bedrock_elevation,
                        ice_thickness, sl0, sl1, dg0, dg1)
    return jnp.concatenate([dhdt, dsdt])


# speedup vs baseline: 1.2896x; 1.0062x over previous
import functools

import jax
import jax.numpy as jnp
from jax import lax
from jax.experimental import pallas as pl
from jax.experimental.pallas import tpu as pltpu
from jax.experimental.pallas import tpu_sc as plsc

N = 100000
E = 3200000
SHEET_COND = 0.01
SHEET_EXP = 1.25
CHAN_COND = 0.1
CHAN_EXP = 3.0
BED_STEP = 0.1
CAV_SPACING = 2.0
CLOSURE = 5e-25
PMC = 7.5e-08
CP = 4220.0
RHO_W = 1000.0
RHO_I = 917.0
G = 9.81
SEC_PER_A = 31556926.0
LATENT = 334000.0
RWG = RHO_W * G
RIG = RHO_I * G

NC = 2
NS = 16
NW = NC * NS

CH = 128
RB = 4
UE = RB * CH
N_UNITS = E // UE
BASE_UNITS = N_UNITS // NW
EXTRA = N_UNITS - BASE_UNITS * NW

NSL = 6240
NTAIL = N - NS * NSL
NTB = NS * NSL

WSL = 3120
WTAIL = N - NW * WSL
WTB = NW * WSL


def _rsqrt(x):
    i = lax.bitcast_convert_type(x, jnp.int32)
    i = jnp.int32(0x5F3759DF) - (i >> 1)
    y = lax.bitcast_convert_type(i, jnp.float32)
    for _ in range(2):
        y = y * (1.5 - 0.5 * x * y * y)
    return y


def _edge_math(pth, ptt, hne_h, hne_t, wph, wpt, s_ch, u_sl):
    gh = lax.bitcast_convert_type(hne_h, jnp.int32)
    gt = lax.bitcast_convert_type(hne_t, jnp.int32)
    hh = lax.bitcast_convert_type(gh << 16, jnp.float32)
    ht = lax.bitcast_convert_type(gt << 16, jnp.float32)
    neh = lax.bitcast_convert_type(gh & jnp.int32(-65536), jnp.float32)
    net = lax.bitcast_convert_type(gt & jnp.int32(-65536), jnp.float32)
    grad = pth - ptt
    absg = jnp.abs(grad) + 1e-8
    hl = 0.5 * (hh + ht)
    hs = jnp.maximum(hl, 1e-30)
    hp = hl * _rsqrt(_rsqrt(hs))
    sheet_q = (-SHEET_COND) * hp * _rsqrt(absg) * grad
    chan_q = (-CHAN_COND) * (s_ch * s_ch * s_ch) * grad
    diss = jnp.abs(chan_q * grad) + jnp.abs(CAV_SPACING * sheet_q * grad)
    pgrad = wph - wpt
    cond = (s_ch > 0) | ((pgrad * sheet_q) > 0)
    totq = jnp.where(cond, chan_q + CAV_SPACING * sheet_q, chan_q)
    sens = (-PMC * CP * RHO_W) * totq * pgrad
    nl = 0.5 * (neh + net)
    nlc = jnp.maximum(nl, 0.0)
    ccl = CLOSURE * s_ch * (nlc * nlc * nlc)
    melt = (diss - sens) * (1.0 / (RHO_I * LATENT))
    dsdt = melt - ccl
    aslide = jnp.abs(u_sl) * (1.0 / SEC_PER_A)
    return dsdt, aslide


def _edge_body(pot_hbm, h_hbm, bed_hbm, ice_hbm, chan_hbm, slid_hbm, ei_hbm,
               dq_out, sl0_out, sl1_out, dg0_out, dg1_out,
               hne_sh, wp_sh, slide_sh, deg_sh,
               b1, b2, b3,
               ihb, itb, sv, uv, gph, gpt, ghh, ght, gwh, gwt, dq, sl, ones,
               seml, semg, semh, sems):
    c = lax.axis_index("c")
    s = lax.axis_index("s")
    w = s * NC + c

    def _stage(nb, nsl, iters):
        pltpu.sync_copy(pot_hbm.at[pl.ds(nb, nsl)], b3.at[pl.ds(0, nsl)])
        pltpu.sync_copy(bed_hbm.at[pl.ds(nb, nsl)], b1.at[pl.ds(0, nsl)])
        pltpu.sync_copy(ice_hbm.at[pl.ds(nb, nsl)], b2.at[pl.ds(0, nsl)])

        def nbody(i, carry):
            dsl = pl.ds(pl.multiple_of(i * 16, 16), 16)
            p = b3[dsl]
            bp = RWG * b1[dsl]
            icv = b2[dsl]
            b1[dsl] = p - bp
            b2[dsl] = bp + RIG * icv - p
            return carry

        lax.fori_loop(0, iters, nbody, 0)
        pltpu.sync_copy(b1.at[pl.ds(0, nsl)], wp_sh.at[pl.ds(nb, nsl)])
        pltpu.sync_copy(h_hbm.at[pl.ds(nb, nsl)], b3.at[pl.ds(0, nsl)])

        def pbody(i, carry):
            dsl = pl.ds(pl.multiple_of(i * 16, 16), 16)
            neb = lax.bitcast_convert_type(b2[dsl], jnp.int32)
            hb = lax.bitcast_convert_type(b3[dsl], jnp.int32)
            pk = (neb & jnp.int32(-65536)) | (hb >> 16)
            b2[dsl] = lax.bitcast_convert_type(pk, jnp.float32)
            return carry

        lax.fori_loop(0, iters, pbody, 0)
        pltpu.sync_copy(b2.at[pl.ds(0, nsl)], hne_sh.at[pl.ds(nb, nsl)])

        def zbody(i, carry):
            dsl = pl.ds(pl.multiple_of(i * 16, 16), 16)
            b1[dsl] = jnp.zeros((16,), jnp.float32)
            return carry

        lax.fori_loop(0, iters, zbody, 0)
        pltpu.sync_copy(b1.at[pl.ds(0, nsl)], slide_sh.at[pl.ds(nb, nsl)])
        pltpu.sync_copy(b1.at[pl.ds(0, nsl)], deg_sh.at[pl.ds(nb, nsl)])

    _stage(pl.multiple_of(s * NSL, 32), NSL, NSL // 16)

    @pl.when(s == 0)
    def _():
        _stage(NTB, NTAIL, NTAIL // 16)

    for i in range(UE // 16):
        ones[pl.ds(i * 16, 16)] = jnp.ones((16,), jnp.float32)

    plsc.subcore_barrier()

    n_iters = jnp.where(w < EXTRA, BASE_UNITS + 1, BASE_UNITS)
    start = BASE_UNITS * w + jnp.minimum(w, EXTRA)

    def _po(p):
        return pl.ds(pl.multiple_of(p * UE, UE), UE)

    def _fire_linear(j, p):
        base = pl.multiple_of((start + j) * UE, UE)
        pltpu.async_copy(chan_hbm.at[pl.ds(base, UE)], sv.at[_po(p)], seml)
        pltpu.async_copy(slid_hbm.at[pl.ds(base, UE)], uv.at[_po(p)], seml)
        pltpu.async_copy(ei_hbm.at[pl.ds(E + base, UE)], ihb.at[_po(p)], seml)
        pltpu.async_copy(ei_hbm.at[pl.ds(base, UE)], itb.at[_po(p)], seml)

    def _drain_linear(p):
        pltpu.make_async_copy(chan_hbm.at[pl.ds(0, UE)], sv.at[_po(p)], seml).wait()
        pltpu.make_async_copy(slid_hbm.at[pl.ds(0, UE)], uv.at[_po(p)], seml).wait()
        pltpu.make_async_copy(ei_hbm.at[pl.ds(0, UE)], ihb.at[_po(p)], seml).wait()
        pltpu.make_async_copy(ei_hbm.at[pl.ds(0, UE)], itb.at[_po(p)], seml).wait()

    def _gather_list(p):
        ihr = ihb.at[_po(p)]
        itr = itb.at[_po(p)]
        hb = [(pot_hbm.at[ihr], gph.at[_po(p)]),
              (pot_hbm.at[itr], gpt.at[_po(p)])]
        sp = [(hne_sh.at[ihr], ghh.at[_po(p)]),
              (hne_sh.at[itr], ght.at[_po(p)]),
              (wp_sh.at[ihr], gwh.at[_po(p)]),
              (wp_sh.at[itr], gwt.at[_po(p)])]
        return sp, hb

    def _fire_gathers(p):
        sp, hb = _gather_list(p)
        for src, dst in hb:
            pltpu.async_copy(src, dst, semh.at[p])
        for src, dst in sp:
            pltpu.async_copy(src, dst, semg.at[p])

    def _drain_gathers(p):
        sp, hb = _gather_list(p)
        for src, dst in sp:
            pltpu.make_async_copy(src, dst, semg.at[p]).wait()
        for src, dst in hb:
            pltpu.make_async_copy(src, dst, semh.at[p]).wait()

    _fire_linear(0, 0)
    _drain_linear(0)
    _fire_gathers(0)
    _fire_linear(1, 1)

    def ebody(j, carry):
        p = lax.rem(j, 2)
        q = 1 - p
        base = pl.multiple_of((start + j) * UE, UE)

        @pl.when(j + 1 < n_iters)
        def _():
            _drain_linear(q)
            _fire_gathers(q)

        _drain_gathers(p)
        pb = pl.multiple_of(p * UE, UE)
        for i in range(UE // 16):
            dsl = pl.ds(pb + i * 16, 16)
            dsdt, aslide = _edge_math(gph[dsl], gpt[dsl],
                                      ghh[dsl], ght[dsl],
                                      gwh[dsl], gwt[dsl],
                                      sv[dsl], uv[dsl])
            dq[dsl] = dsdt
            sl[dsl] = aslide
        pltpu.async_copy(sl.at[_po(p)], slide_sh.at[ihb.at[_po(p)]], sems, add=True)
        pltpu.async_copy(sl.at[_po(p)], slide_sh.at[itb.at[_po(p)]], sems, add=True)
        pltpu.async_copy(ones, deg_sh.at[ihb.at[_po(p)]], sems, add=True)
        pltpu.async_copy(ones, deg_sh.at[itb.at[_po(p)]], sems, add=True)
        ocp = pltpu.async_copy(dq.at[_po(p)], dq_out.at[pl.ds(base, UE)], seml)
        pltpu.make_async_copy(sl.at[_po(p)], slide_sh.at[ihb.at[_po(p)]], sems).wait()
        pltpu.make_async_copy(sl.at[_po(p)], slide_sh.at[itb.at[_po(p)]], sems).wait()
        pltpu.make_async_copy(ones, deg_sh.at[ihb.at[_po(p)]], sems).wait()
        pltpu.make_async_copy(ones, deg_sh.at[itb.at[_po(p)]], sems).wait()
        ocp.wait()

        @pl.when(j + 2 < n_iters)
        def _():
            _fire_linear(j + 2, p)

        return carry

    lax.fori_loop(0, n_iters, ebody, 0)

    plsc.subcore_barrier()

    def _wb(nb, nsl, slide_out, deg_out):
        pltpu.sync_copy(slide_sh.at[pl.ds(nb, nsl)], b1.at[pl.ds(0, nsl)])
        pltpu.sync_copy(b1.at[pl.ds(0, nsl)], slide_out.at[pl.ds(nb, nsl)])
        pltpu.sync_copy(deg_sh.at[pl.ds(nb, nsl)], b2.at[pl.ds(0, nsl)])
        pltpu.sync_copy(b2.at[pl.ds(0, nsl)], deg_out.at[pl.ds(nb, nsl)])

    nb_main = pl.multiple_of(s * NSL, 32)

    @pl.when(c == 0)
    def _():
        _wb(nb_main, NSL, sl0_out, dg0_out)

    @pl.when(c == 1)
    def _():
        _wb(nb_main, NSL, sl1_out, dg1_out)

    @pl.when((s == 0) & (c == 0))
    def _():
        _wb(NTB, NTAIL, sl0_out, dg0_out)

    @pl.when((s == 0) & (c == 1))
    def _():
        _wb(NTB, NTAIL, sl1_out, dg1_out)


def _node_body(pot_hbm, h_hbm, bed_hbm, ice_hbm, sl0_hbm, sl1_hbm, dg0_hbm, dg1_hbm,
               dh_out,
               potb, hb, bedb, iceb, sp0, sp1, dp0, dp1, dhb):
    c = lax.axis_index("c")
    s = lax.axis_index("s")
    w = s * NC + c

    def _run(nb, nsl, iters):
        pltpu.sync_copy(pot_hbm.at[pl.ds(nb, nsl)], potb.at[pl.ds(0, nsl)])
        pltpu.sync_copy(h_hbm.at[pl.ds(nb, nsl)], hb.at[pl.ds(0, nsl)])
        pltpu.sync_copy(bed_hbm.at[pl.ds(nb, nsl)], bedb.at[pl.ds(0, nsl)])
        pltpu.sync_copy(ice_hbm.at[pl.ds(nb, nsl)], iceb.at[pl.ds(0, nsl)])
        pltpu.sync_copy(sl0_hbm.at[pl.ds(nb, nsl)], sp0.at[pl.ds(0, nsl)])
        pltpu.sync_copy(sl1_hbm.at[pl.ds(nb, nsl)], sp1.at[pl.ds(0, nsl)])
        pltpu.sync_copy(dg0_hbm.at[pl.ds(nb, nsl)], dp0.at[pl.ds(0, nsl)])
        pltpu.sync_copy(dg1_hbm.at[pl.ds(nb, nsl)], dp1.at[pl.ds(0, nsl)])

        def nbody(i, carry):
            dsl = pl.ds(pl.multiple_of(i * 16, 16), 16)
            p = potb[dsl]
            h = hb[dsl]
            ne = RWG * bedb[dsl] + RIG * iceb[dsl] - p
            nec = jnp.maximum(ne, 0.0)
            scl = CLOSURE * h * (nec * nec * nec)
            dg = dp0[dsl] + dp1[dsl]
            sn = (sp0[dsl] + sp1[dsl]) / jnp.maximum(dg, 1.0)
            opening = jnp.where(h < BED_STEP,
                                sn * (BED_STEP - h) * (1.0 / CAV_SPACING), 0.0)
            dhb[dsl] = opening - scl
            return carry

        lax.fori_loop(0, iters, nbody, 0)
        pltpu.sync_copy(dhb.at[pl.ds(0, nsl)], dh_out.at[pl.ds(nb, nsl)])

    _run(pl.multiple_of(w * WSL, 16), WSL, WSL // 16)

    @pl.when(w == 0)
    def _():
        _run(WTB, WTAIL, WTAIL // 16)


_MESH = plsc.VectorSubcoreMesh(core_axis_name="c", subcore_axis_name="s")

_edge_kernel = functools.partial(
    pl.kernel,
    out_type=(jax.ShapeDtypeStruct((E,), jnp.float32),
              jax.ShapeDtypeStruct((N,), jnp.float32),
              jax.ShapeDtypeStruct((N,), jnp.float32),
              jax.ShapeDtypeStruct((N,), jnp.float32),
              jax.ShapeDtypeStruct((N,), jnp.float32)),
    mesh=_MESH,
    scratch_types=(
        pltpu.VMEM_SHARED((N,), jnp.float32),
        pltpu.VMEM_SHARED((N,), jnp.float32),
        pltpu.VMEM_SHARED((N,), jnp.float32),
        pltpu.VMEM_SHARED((N,), jnp.float32),
        pltpu.VMEM((NSL,), jnp.float32),
        pltpu.VMEM((NSL,), jnp.float32),
        pltpu.VMEM((NSL,), jnp.float32),
        pltpu.VMEM((2 * UE,), jnp.int32),
        pltpu.VMEM((2 * UE,), jnp.int32),
        pltpu.VMEM((2 * UE,), jnp.float32),
        pltpu.VMEM((2 * UE,), jnp.float32),
        pltpu.VMEM((2 * UE,), jnp.float32),
        pltpu.VMEM((2 * UE,), jnp.float32),
        pltpu.VMEM((2 * UE,), jnp.float32),
        pltpu.VMEM((2 * UE,), jnp.float32),
        pltpu.VMEM((2 * UE,), jnp.float32),
        pltpu.VMEM((2 * UE,), jnp.float32),
        pltpu.VMEM((2 * UE,), jnp.float32),
        pltpu.VMEM((2 * UE,), jnp.float32),
        pltpu.VMEM((UE,), jnp.float32),
        pltpu.SemaphoreType.DMA,
        pltpu.SemaphoreType.DMA((2,)),
        pltpu.SemaphoreType.DMA((2,)),
        pltpu.SemaphoreType.DMA,
    ),
)(_edge_body)

_node_kernel = functools.partial(
    pl.kernel,
    out_type=jax.ShapeDtypeStruct((N,), jnp.float32),
    mesh=_MESH,
    scratch_types=tuple([pltpu.VMEM((WSL,), jnp.float32)] * 9),
)(_node_body)


def kernel(potential, sheet_thickness, channel_size, sliding_velocity,
           bedrock_elevation, ice_thickness, edge_index):
    ei = edge_index.reshape(-1)
    dsdt, sl0, sl1, dg0, dg1 = _edge_kernel(
        potential, sheet_thickness, bedrock_elevation, ice_thickness,
        channel_size, sliding_velocity, ei)
    dhdt = _node_kernel(potential, sheet_thickness, bedrock_elevation,
                        ice_thickness, sl0, sl1, dg0, dg1)
    return jnp.concatenate([dhdt, dsdt])
